# baseline (speedup 1.0000x reference)
import jax
import jax.numpy as jnp
from jax.experimental import pallas as pl
from jax.experimental.pallas import tpu as pltpu

_BI = 1024
_BJ = 1024


def _ahl_kernel(x_ref, y_ref, out_ref, row_acc, col_acc):
    i = pl.program_id(0)
    j = pl.program_id(1)
    ni = pl.num_programs(0)
    nj = pl.num_programs(1)

    x = x_ref[...]
    y = y_ref[...]
    xy = jax.lax.dot_general(
        x, y, (((1,), (1,)), ((), ())),
        preferred_element_type=jnp.float32,
        precision=jax.lax.Precision.HIGHEST,
    )
    x2 = jnp.sum(x * x, axis=1, keepdims=True)
    y2 = jnp.sum(y * y, axis=1, keepdims=True)
    d2 = (x2 - 2.0 * xy) + y2.T

    row_part = jnp.min(d2, axis=1, keepdims=True)
    col_part = jnp.min(d2, axis=0, keepdims=True)

    @pl.when(j == 0)
    def _():
        row_acc[...] = row_part

    @pl.when(j != 0)
    def _():
        row_acc[...] = jnp.minimum(row_acc[...], row_part)

    csl = pl.ds(j * _BJ, _BJ)

    @pl.when(i == 0)
    def _():
        col_acc[:, csl] = col_part

    @pl.when(i != 0)
    def _():
        col_acc[:, csl] = jnp.minimum(col_acc[:, csl], col_part)

    @pl.when(jnp.logical_and(i == 0, j == 0))
    def _():
        out_ref[...] = jnp.zeros((1, 1), jnp.float32)

    @pl.when(j == nj - 1)
    def _():
        r = jnp.sqrt(jnp.maximum(row_acc[...], 1e-12))
        out_ref[...] += jnp.sum(r).reshape(1, 1) / (ni * _BI)

    @pl.when(jnp.logical_and(i == ni - 1, j == nj - 1))
    def _():
        c = jnp.sqrt(jnp.maximum(col_acc[...], 1e-12))
        out_ref[...] += jnp.sum(c).reshape(1, 1) / (nj * _BJ)


@jax.jit
def kernel(set1, set2):
    s1 = set1.reshape(-1, set1.shape[-1])
    s2 = set2.reshape(-1, set2.shape[-1])
    n, d = s1.shape
    m = s2.shape[0]
    out = pl.pallas_call(
        _ahl_kernel,
        grid=(n // _BI, m // _BJ),
        in_specs=[
            pl.BlockSpec((_BI, d), lambda i, j: (i, 0)),
            pl.BlockSpec((_BJ, d), lambda i, j: (j, 0)),
        ],
        out_specs=pl.BlockSpec((1, 1), lambda i, j: (0, 0)),
        out_shape=jax.ShapeDtypeStruct((1, 1), jnp.float32),
        scratch_shapes=[
            pltpu.VMEM((_BI, 1), jnp.float32),
            pltpu.VMEM((1, m), jnp.float32),
        ],
    )(s1, s2)
    return out[0, 0]


# speedup vs baseline: 2.5913x; 2.5913x over previous
import jax
import jax.numpy as jnp
from jax.experimental import pallas as pl
from jax.experimental.pallas import tpu as pltpu

_BI = 1024
_BJ = 1024


def _ahl_kernel(x_ref, y_ref, out_ref, row_acc, col_acc):
    i = pl.program_id(0)
    j = pl.program_id(1)
    ni = pl.num_programs(0)
    nj = pl.num_programs(1)

    x = x_ref[...]
    y = y_ref[...]
    d2 = jax.lax.dot_general(
        x, y, (((1,), (1,)), ((), ())),
        preferred_element_type=jnp.float32,
        precision=jax.lax.Precision.DEFAULT,
    )

    row_part = jnp.min(d2, axis=1, keepdims=True)
    col_part = jnp.min(d2, axis=0, keepdims=True)

    @pl.when(j == 0)
    def _():
        row_acc[...] = row_part

    @pl.when(j != 0)
    def _():
        row_acc[...] = jnp.minimum(row_acc[...], row_part)

    csl = pl.ds(j * _BJ, _BJ)

    @pl.when(i == 0)
    def _():
        col_acc[:, csl] = col_part

    @pl.when(i != 0)
    def _():
        col_acc[:, csl] = jnp.minimum(col_acc[:, csl], col_part)

    @pl.when(jnp.logical_and(i == 0, j == 0))
    def _():
        out_ref[...] = jnp.zeros((1, 1), jnp.float32)

    @pl.when(j == nj - 1)
    def _():
        r = jnp.sqrt(jnp.maximum(row_acc[...], 1e-12))
        out_ref[...] += jnp.sum(r).reshape(1, 1) / (ni * _BI)

    @pl.when(jnp.logical_and(i == ni - 1, j == nj - 1))
    def _():
        c = jnp.sqrt(jnp.maximum(col_acc[...], 1e-12))
        out_ref[...] += jnp.sum(c).reshape(1, 1) / (nj * _BJ)


@jax.jit
def kernel(set1, set2):
    s1 = set1.reshape(-1, set1.shape[-1])
    s2 = set2.reshape(-1, set2.shape[-1])
    n = s1.shape[0]
    m = s2.shape[0]
    x2 = jnp.sum(s1 * s1, axis=1, keepdims=True)
    y2 = jnp.sum(s2 * s2, axis=1, keepdims=True)
    ones_n = jnp.ones((n, 1), jnp.float32)
    ones_m = jnp.ones((m, 1), jnp.float32)
    s1 = jnp.concatenate([-2.0 * s1, ones_n, x2], axis=1)
    s2 = jnp.concatenate([s2, y2, ones_m], axis=1)
    d = s1.shape[1]
    out = pl.pallas_call(
        _ahl_kernel,
        grid=(n // _BI, m // _BJ),
        in_specs=[
            pl.BlockSpec((_BI, d), lambda i, j: (i, 0)),
            pl.BlockSpec((_BJ, d), lambda i, j: (j, 0)),
        ],
        out_specs=pl.BlockSpec((1, 1), lambda i, j: (0, 0)),
        out_shape=jax.ShapeDtypeStruct((1, 1), jnp.float32),
        scratch_shapes=[
            pltpu.VMEM((_BI, 1), jnp.float32),
            pltpu.VMEM((1, m), jnp.float32),
        ],
    )(s1, s2)
    return out[0, 0]


# speedup vs baseline: 3.0053x; 1.1598x over previous
import jax
import jax.numpy as jnp
from jax.experimental import pallas as pl
from jax.experimental.pallas import tpu as pltpu

_BI = 1024
_BJ = 1024


def _ahl_kernel(x_ref, y_ref, out_ref, row_acc, col_acc):
    i = pl.program_id(0)
    j = pl.program_id(1)
    ni = pl.num_programs(0)
    nj = pl.num_programs(1)

    x = x_ref[...]
    y = y_ref[...]
    d2 = jax.lax.dot_general(
        x, y, (((1,), (1,)), ((), ())),
        preferred_element_type=jnp.float32,
        precision=jax.lax.Precision.DEFAULT,
    )

    row_part = jnp.min(d2, axis=1, keepdims=True)
    col_part = jnp.min(d2, axis=0, keepdims=True)

    @pl.when(j == 0)
    def _():
        row_acc[...] = row_part

    @pl.when(j != 0)
    def _():
        row_acc[...] = jnp.minimum(row_acc[...], row_part)

    csl = pl.ds(j * _BJ, _BJ)

    @pl.when(i == 0)
    def _():
        col_acc[:, csl] = col_part

    @pl.when(i != 0)
    def _():
        col_acc[:, csl] = jnp.minimum(col_acc[:, csl], col_part)

    @pl.when(jnp.logical_and(i == 0, j == 0))
    def _():
        out_ref[...] = jnp.zeros((1, 1), jnp.float32)

    @pl.when(j == nj - 1)
    def _():
        r = jnp.sqrt(jnp.maximum(row_acc[...], 1e-12))
        out_ref[...] += jnp.sum(r).reshape(1, 1) / (ni * _BI)

    @pl.when(jnp.logical_and(i == ni - 1, j == nj - 1))
    def _():
        c = jnp.sqrt(jnp.maximum(col_acc[...], 1e-12))
        out_ref[...] += jnp.sum(c).reshape(1, 1) / (nj * _BJ)


@jax.jit
def kernel(set1, set2):
    s1 = set1.reshape(-1, set1.shape[-1])
    s2 = set2.reshape(-1, set2.shape[-1])
    n = s1.shape[0]
    m = s2.shape[0]
    x2 = jnp.sum(s1 * s1, axis=1, keepdims=True)
    y2 = jnp.sum(s2 * s2, axis=1, keepdims=True)
    ones_n = jnp.ones((n, 1), jnp.float32)
    ones_m = jnp.ones((m, 1), jnp.float32)
    s1 = jnp.concatenate([-2.0 * s1, ones_n, x2], axis=1).astype(jnp.bfloat16)
    s2 = jnp.concatenate([s2, y2, ones_m], axis=1).astype(jnp.bfloat16)
    d = s1.shape[1]
    out = pl.pallas_call(
        _ahl_kernel,
        grid=(n // _BI, m // _BJ),
        in_specs=[
            pl.BlockSpec((_BI, d), lambda i, j: (i, 0)),
            pl.BlockSpec((_BJ, d), lambda i, j: (j, 0)),
        ],
        out_specs=pl.BlockSpec((1, 1), lambda i, j: (0, 0)),
        out_shape=jax.ShapeDtypeStruct((1, 1), jnp.float32),
        scratch_shapes=[
            pltpu.VMEM((_BI, 1), jnp.float32),
            pltpu.VMEM((1, m), jnp.float32),
        ],
    )(s1, s2)
    return out[0, 0]
